# baseline (speedup 1.0000x reference)
import functools

import jax
import jax.numpy as jnp
from jax import lax
from jax.experimental import pallas as pl
from jax.experimental.pallas import tpu as pltpu
from jax.experimental.pallas import tpu_sc as plsc

_LAMBDA_2 = 0.01

_B = 16384
_D = 64
_V = 100000
_L = 16
_NC = 2
_NS = 16
_NW = _NC * _NS
_DPW = _D // _NW
_NBLK = 4
_BLK = _B // _NBLK
_OUTW = 128
_UNROLL = 8
_NACC = 2

_mesh = plsc.VectorSubcoreMesh(core_axis_name="c", subcore_axis_name="s")


@functools.partial(
    pl.kernel,
    mesh=_mesh,
    compiler_params=pltpu.CompilerParams(needs_layout_passes=False),
    out_type=jax.ShapeDtypeStruct((2, _NW, _OUTW), jnp.float32),
    scratch_types=[
        pltpu.VMEM((_V,), jnp.float32),
        pltpu.VMEM((2, _BLK), jnp.int32),
        pltpu.VMEM((2, _BLK), jnp.int32),
        pltpu.VMEM((2, _BLK), jnp.float32),
        pltpu.VMEM((_OUTW,), jnp.float32),
        pltpu.VMEM((_OUTW,), jnp.float32),
        pltpu.SemaphoreType.DMA,
        [pltpu.SemaphoreType.DMA] * 2,
    ],
)
def _phi_partials(wt_hbm, coo_hbm, idx1_hbm, idx2_hbm, out_hbm,
                  row_v, idx1_v, idx2_v, coo_v, sq_v, abs_v, rowsem, sems):
    wid = lax.axis_index("s") * _NC + lax.axis_index("c")

    def fire_blk(blk):
        buf = blk % 2
        return (
            pltpu.async_copy(idx1_hbm.at[pl.ds(blk * _BLK, _BLK)],
                             idx1_v.at[buf], sems[buf]),
            pltpu.async_copy(idx2_hbm.at[pl.ds(blk * _BLK, _BLK)],
                             idx2_v.at[buf], sems[buf]),
            pltpu.async_copy(coo_hbm.at[pl.ds(blk * _BLK, _BLK)],
                             coo_v.at[buf], sems[buf]),
        )

    zero = jnp.zeros((_L,), jnp.float32)
    accs = tuple((zero, zero) for _ in range(_NACC))
    for p in range(_DPW):
        d = wid * _DPW + p
        row_cp = pltpu.async_copy(wt_hbm.at[d], row_v, rowsem)
        pending = {0: fire_blk(0), 1: fire_blk(1)}
        for blk in range(_NBLK):
            buf = blk % 2
            for cp in pending.pop(blk):
                cp.wait()
            if blk == 0:
                row_cp.wait()

            def body(i, carry, buf=buf):
                out = []
                for u, (a_sq, a_abs) in enumerate(carry):
                    off = pl.ds(i + u * _L, _L)
                    i1 = idx1_v[buf, off]
                    i2 = idx2_v[buf, off]
                    cvec = coo_v[buf, off]
                    g1 = plsc.load_gather(row_v, [i1])
                    g2 = plsc.load_gather(row_v, [i2])
                    dd = cvec - g1 * g2
                    out.append((a_sq + dd * dd,
                                a_abs + jnp.abs(g1) + jnp.abs(g2)))
                return tuple(out)

            accs = plsc.parallel_loop(
                0, _BLK, step=_L * _NACC, unroll=_UNROLL,
                carry=accs)(body)
            if blk + 2 < _NBLK:
                pending[blk + 2] = fire_blk(blk + 2)

    acc_sq = zero
    acc_abs = zero
    for a_sq, a_abs in accs:
        acc_sq = acc_sq + a_sq
        acc_abs = acc_abs + a_abs

    for t in range(_OUTW // _L):
        sq_v[pl.ds(t * _L, _L)] = acc_sq if t == 0 else zero
        abs_v[pl.ds(t * _L, _L)] = acc_abs if t == 0 else zero
    pltpu.sync_copy(sq_v, out_hbm.at[0, wid])
    pltpu.sync_copy(abs_v, out_hbm.at[1, wid])


def kernel(w, cooccur, feature_idx1, feature_idx2):
    wt = w.T
    idx1 = feature_idx1.astype(jnp.int32)
    idx2 = feature_idx2.astype(jnp.int32)
    coo = cooccur.reshape(_B)
    parts = _phi_partials(wt, coo, idx1, idx2)
    return (jnp.sqrt(jnp.sum(parts[0]))
            + (_LAMBDA_2 / 2.0) * jnp.sum(parts[1]))


# speedup vs baseline: 1.0364x; 1.0364x over previous
import functools

import jax
import jax.numpy as jnp
from jax import lax
from jax.experimental import pallas as pl
from jax.experimental.pallas import tpu as pltpu
from jax.experimental.pallas import tpu_sc as plsc

_LAMBDA_2 = 0.01

_B = 16384
_D = 64
_V = 100000
_L = 16
_NC = 2
_NS = 16
_NW = _NC * _NS
_DPW = _D // _NW
_NBLK = 4
_BLK = _B // _NBLK
_OUTW = 128
_UNROLL = 4
_NACC = 2

_mesh = plsc.VectorSubcoreMesh(core_axis_name="c", subcore_axis_name="s")


@functools.partial(
    pl.kernel,
    mesh=_mesh,
    compiler_params=pltpu.CompilerParams(needs_layout_passes=False),
    out_type=jax.ShapeDtypeStruct((2, _NW, _OUTW), jnp.float32),
    scratch_types=[
        pltpu.VMEM((_V,), jnp.float32),
        pltpu.VMEM((2, _BLK), jnp.int32),
        pltpu.VMEM((2, _BLK), jnp.int32),
        pltpu.VMEM((2, _BLK), jnp.float32),
        pltpu.VMEM((_OUTW,), jnp.float32),
        pltpu.VMEM((_OUTW,), jnp.float32),
        pltpu.SemaphoreType.DMA,
        [pltpu.SemaphoreType.DMA] * 2,
    ],
)
def _phi_partials(wt_hbm, coo_hbm, idx1_hbm, idx2_hbm, out_hbm,
                  row_v, idx1_v, idx2_v, coo_v, sq_v, abs_v, rowsem, sems):
    wid = lax.axis_index("s") * _NC + lax.axis_index("c")

    def fire_blk(blk):
        buf = blk % 2
        return (
            pltpu.async_copy(idx1_hbm.at[pl.ds(blk * _BLK, _BLK)],
                             idx1_v.at[buf], sems[buf]),
            pltpu.async_copy(idx2_hbm.at[pl.ds(blk * _BLK, _BLK)],
                             idx2_v.at[buf], sems[buf]),
            pltpu.async_copy(coo_hbm.at[pl.ds(blk * _BLK, _BLK)],
                             coo_v.at[buf], sems[buf]),
        )

    zero = jnp.zeros((_L,), jnp.float32)
    accs = tuple((zero, zero) for _ in range(_NACC))
    for p in range(_DPW):
        d = wid * _DPW + p
        row_cp = pltpu.async_copy(wt_hbm.at[d], row_v, rowsem)
        pending = {0: fire_blk(0), 1: fire_blk(1)}
        for blk in range(_NBLK):
            buf = blk % 2
            for cp in pending.pop(blk):
                cp.wait()
            if blk == 0:
                row_cp.wait()

            def body(i, carry, buf=buf):
                out = []
                for u, (a_sq, a_abs) in enumerate(carry):
                    off = pl.ds(i + u * _L, _L)
                    i1 = idx1_v[buf, off]
                    i2 = idx2_v[buf, off]
                    cvec = coo_v[buf, off]
                    g1 = plsc.load_gather(row_v, [i1])
                    g2 = plsc.load_gather(row_v, [i2])
                    dd = cvec - g1 * g2
                    out.append((a_sq + dd * dd,
                                a_abs + jnp.abs(g1) + jnp.abs(g2)))
                return tuple(out)

            accs = plsc.parallel_loop(
                0, _BLK, step=_L * _NACC, unroll=_UNROLL,
                carry=accs)(body)
            if blk + 2 < _NBLK:
                pending[blk + 2] = fire_blk(blk + 2)

    acc_sq = zero
    acc_abs = zero
    for a_sq, a_abs in accs:
        acc_sq = acc_sq + a_sq
        acc_abs = acc_abs + a_abs

    for t in range(_OUTW // _L):
        sq_v[pl.ds(t * _L, _L)] = acc_sq if t == 0 else zero
        abs_v[pl.ds(t * _L, _L)] = acc_abs if t == 0 else zero
    pltpu.sync_copy(sq_v, out_hbm.at[0, wid])
    pltpu.sync_copy(abs_v, out_hbm.at[1, wid])


def kernel(w, cooccur, feature_idx1, feature_idx2):
    wt = w.T
    idx1 = feature_idx1.astype(jnp.int32)
    idx2 = feature_idx2.astype(jnp.int32)
    coo = cooccur.reshape(_B)
    parts = _phi_partials(wt, coo, idx1, idx2)
    return (jnp.sqrt(jnp.sum(parts[0]))
            + (_LAMBDA_2 / 2.0) * jnp.sum(parts[1]))


# speedup vs baseline: 1.0570x; 1.0199x over previous
import functools

import jax
import jax.numpy as jnp
from jax import lax
from jax.experimental import pallas as pl
from jax.experimental.pallas import tpu as pltpu
from jax.experimental.pallas import tpu_sc as plsc

_LAMBDA_2 = 0.01

_B = 16384
_D = 64
_V = 100000
_L = 16
_NC = 2
_NS = 16
_NW = _NC * _NS
_DPW = _D // _NW
_NBLK = 4
_BLK = _B // _NBLK
_OUTW = 128
_UNROLL = 4
_NACC = 2

_mesh = plsc.VectorSubcoreMesh(core_axis_name="c", subcore_axis_name="s")


@functools.partial(
    pl.kernel,
    mesh=_mesh,
    compiler_params=pltpu.CompilerParams(needs_layout_passes=False),
    out_type=jax.ShapeDtypeStruct((2, _NW, _OUTW), jnp.float32),
    scratch_types=[
        pltpu.VMEM((_V,), jnp.float32),
        pltpu.VMEM((2, _BLK), jnp.int32),
        pltpu.VMEM((2, _BLK), jnp.int32),
        pltpu.VMEM((2, _BLK), jnp.float32),
        pltpu.VMEM((_OUTW,), jnp.float32),
        pltpu.VMEM((_OUTW,), jnp.float32),
        pltpu.SemaphoreType.DMA,
        [pltpu.SemaphoreType.DMA] * 2,
    ],
)
def _phi_partials(wt_hbm, coo_hbm, idx1_hbm, idx2_hbm, out_hbm,
                  row_v, idx1_v, idx2_v, coo_v, sq_v, abs_v, rowsem, sems):
    wid = lax.axis_index("c") * _NS + lax.axis_index("s")

    def fire_blk(blk):
        buf = blk % 2
        return (
            pltpu.async_copy(idx1_hbm.at[pl.ds(blk * _BLK, _BLK)],
                             idx1_v.at[buf], sems[buf]),
            pltpu.async_copy(idx2_hbm.at[pl.ds(blk * _BLK, _BLK)],
                             idx2_v.at[buf], sems[buf]),
            pltpu.async_copy(coo_hbm.at[pl.ds(blk * _BLK, _BLK)],
                             coo_v.at[buf], sems[buf]),
        )

    zero = jnp.zeros((_L,), jnp.float32)
    accs = tuple((zero, zero) for _ in range(_NACC))
    for p in range(_DPW):
        d = wid * _DPW + p
        row_cp = pltpu.async_copy(wt_hbm.at[d], row_v, rowsem)
        pending = {0: fire_blk(0), 1: fire_blk(1)}
        for blk in range(_NBLK):
            buf = blk % 2
            for cp in pending.pop(blk):
                cp.wait()
            if blk == 0:
                row_cp.wait()

            def body(i, carry, buf=buf):
                out = []
                for u, (a_sq, a_abs) in enumerate(carry):
                    off = pl.ds(i + u * _L, _L)
                    i1 = idx1_v[buf, off]
                    i2 = idx2_v[buf, off]
                    cvec = coo_v[buf, off]
                    g1 = plsc.load_gather(row_v, [i1])
                    g2 = plsc.load_gather(row_v, [i2])
                    dd = cvec - g1 * g2
                    out.append((a_sq + dd * dd,
                                a_abs + jnp.abs(g1) + jnp.abs(g2)))
                return tuple(out)

            accs = plsc.parallel_loop(
                0, _BLK, step=_L * _NACC, unroll=_UNROLL,
                carry=accs)(body)
            if blk + 2 < _NBLK:
                pending[blk + 2] = fire_blk(blk + 2)

    acc_sq = zero
    acc_abs = zero
    for a_sq, a_abs in accs:
        acc_sq = acc_sq + a_sq
        acc_abs = acc_abs + a_abs

    for t in range(_OUTW // _L):
        sq_v[pl.ds(t * _L, _L)] = acc_sq if t == 0 else zero
        abs_v[pl.ds(t * _L, _L)] = acc_abs if t == 0 else zero
    pltpu.sync_copy(sq_v, out_hbm.at[0, wid])
    pltpu.sync_copy(abs_v, out_hbm.at[1, wid])


def kernel(w, cooccur, feature_idx1, feature_idx2):
    wt = w.T
    idx1 = feature_idx1.astype(jnp.int32)
    idx2 = feature_idx2.astype(jnp.int32)
    coo = cooccur.reshape(_B)
    parts = _phi_partials(wt, coo, idx1, idx2)
    return (jnp.sqrt(jnp.sum(parts[0]))
            + (_LAMBDA_2 / 2.0) * jnp.sum(parts[1]))
